# baseline (speedup 1.0000x reference)
import jax
import jax.numpy as jnp
from jax.experimental import pallas as pl
from jax.experimental.pallas import tpu as pltpu

N = 16384
D = 128
CHUNKS = [256, 256, 512, 1024, 2048] + [4096] * 3
assert sum(CHUNKS) == N
OFFS = [sum(CHUNKS[:i]) for i in range(len(CHUNKS))]
NCH = len(CHUNKS)
CMAX = max(CHUNKS)
NBUF = 5


def _tc_body(x_hbm, s_ref, o1_hbm, o2_hbm, o3_hbm, ibufs, rbufs, in_sems, out_sems):
    outs = (o1_hbm, o2_hbm, o3_hbm)

    def in_copy(c):
        return pltpu.make_async_copy(
            x_hbm.at[pl.ds(OFFS[c], CHUNKS[c])],
            ibufs.at[c % NBUF, pl.ds(0, CHUNKS[c])],
            in_sems.at[c % NBUF],
        )

    def out_copy(c, k):
        return pltpu.make_async_copy(
            rbufs.at[c % NBUF, pl.ds(0, CHUNKS[c])],
            outs[k].at[pl.ds(OFFS[c], CHUNKS[c])],
            out_sems.at[c % NBUF],
        )

    for c in range(min(NBUF, NCH)):
        in_copy(c).start()

    s = s_ref[0, 0]
    for c in range(NCH):
        if c >= NBUF:
            for k in range(3):
                out_copy(c - NBUF, k).wait()
        in_copy(c).wait()
        x = ibufs[c % NBUF, : CHUNKS[c]]
        valid = jnp.any(x != 0.0, axis=-1, keepdims=True)
        rbufs[c % NBUF, : CHUNKS[c]] = x + jnp.where(valid, s, 0.0)
        for k in range(3):
            out_copy(c, k).start()
        if c + NBUF < NCH:
            in_copy(c + NBUF).start()

    for c in range(NCH - min(NBUF, NCH), NCH):
        for k in range(3):
            out_copy(c, k).wait()


@jax.jit
def _run(inputs, shift_s):
    f = pl.pallas_call(
        _tc_body,
        out_shape=[jax.ShapeDtypeStruct((N, D), jnp.float32)] * 3,
        in_specs=[
            pl.BlockSpec(memory_space=pltpu.MemorySpace.HBM),
            pl.BlockSpec(memory_space=pltpu.SMEM),
        ],
        out_specs=[pl.BlockSpec(memory_space=pltpu.MemorySpace.HBM)] * 3,
        scratch_shapes=[
            pltpu.VMEM((NBUF, CMAX, D), jnp.float32),
            pltpu.VMEM((NBUF, CMAX, D), jnp.float32),
            pltpu.SemaphoreType.DMA((NBUF,)),
            pltpu.SemaphoreType.DMA((NBUF,)),
        ],
    )
    return f(inputs, shift_s)


def kernel(inputs, shift):
    o1, o2, o3 = _run(inputs, jnp.reshape(shift, (1, 1)))
    return (o1, o2, o3)


# speedup vs baseline: 1.0106x; 1.0106x over previous
import jax
import jax.numpy as jnp
from jax.experimental import pallas as pl
from jax.experimental.pallas import tpu as pltpu

N = 16384
D = 128
CHUNKS = [256, 256, 512, 1024] + [2048] * 7
assert sum(CHUNKS) == N
OFFS = [sum(CHUNKS[:i]) for i in range(len(CHUNKS))]
NCH = len(CHUNKS)
CMAX = max(CHUNKS)
NBUF = 5


def _tc_body(x_hbm, s_ref, o1_hbm, o2_hbm, o3_hbm, ibufs, rbufs, in_sems, out_sems):
    outs = (o1_hbm, o2_hbm, o3_hbm)

    def in_copy(c):
        return pltpu.make_async_copy(
            x_hbm.at[pl.ds(OFFS[c], CHUNKS[c])],
            ibufs.at[c % NBUF, pl.ds(0, CHUNKS[c])],
            in_sems.at[c % NBUF],
        )

    def out_copy(c, k):
        return pltpu.make_async_copy(
            rbufs.at[c % NBUF, pl.ds(0, CHUNKS[c])],
            outs[k].at[pl.ds(OFFS[c], CHUNKS[c])],
            out_sems.at[c % NBUF],
        )

    for c in range(min(NBUF, NCH)):
        in_copy(c).start()

    s = s_ref[0, 0]
    for c in range(NCH):
        if c >= NBUF:
            for k in range(3):
                out_copy(c - NBUF, k).wait()
        in_copy(c).wait()
        x = ibufs[c % NBUF, : CHUNKS[c]]
        valid = jnp.any(x != 0.0, axis=-1, keepdims=True)
        rbufs[c % NBUF, : CHUNKS[c]] = x + jnp.where(valid, s, 0.0)
        for k in range(3):
            out_copy(c, k).start()
        if c + NBUF < NCH:
            in_copy(c + NBUF).start()

    for c in range(NCH - min(NBUF, NCH), NCH):
        for k in range(3):
            out_copy(c, k).wait()


@jax.jit
def _run(inputs, shift_s):
    f = pl.pallas_call(
        _tc_body,
        out_shape=[jax.ShapeDtypeStruct((N, D), jnp.float32)] * 3,
        in_specs=[
            pl.BlockSpec(memory_space=pltpu.MemorySpace.HBM),
            pl.BlockSpec(memory_space=pltpu.SMEM),
        ],
        out_specs=[pl.BlockSpec(memory_space=pltpu.MemorySpace.HBM)] * 3,
        scratch_shapes=[
            pltpu.VMEM((NBUF, CMAX, D), jnp.float32),
            pltpu.VMEM((NBUF, CMAX, D), jnp.float32),
            pltpu.SemaphoreType.DMA((NBUF,)),
            pltpu.SemaphoreType.DMA((NBUF,)),
        ],
    )
    return f(inputs, shift_s)


def kernel(inputs, shift):
    o1, o2, o3 = _run(inputs, jnp.reshape(shift, (1, 1)))
    return (o1, o2, o3)


# speedup vs baseline: 1.0135x; 1.0029x over previous
import jax
import jax.numpy as jnp
from jax.experimental import pallas as pl
from jax.experimental.pallas import tpu as pltpu

N = 16384
D = 128
CHUNKS = [256, 256, 512, 1024] + [2048] * 7
assert sum(CHUNKS) == N
OFFS = [sum(CHUNKS[:i]) for i in range(len(CHUNKS))]
NCH = len(CHUNKS)
CMAX = max(CHUNKS)
NBUF = 5


def _tc_body(x_hbm, s_ref, o1_hbm, o2_hbm, o3_hbm, ibufs, rbufs, in_sems, out_sems):
    outs = (o1_hbm, o2_hbm, o3_hbm)

    def in_copy(c):
        return pltpu.make_async_copy(
            x_hbm.at[pl.ds(OFFS[c], CHUNKS[c])],
            ibufs.at[c % NBUF, pl.ds(0, CHUNKS[c])],
            in_sems.at[c % NBUF],
        )

    def out_copies_for(c, k):
        h = CHUNKS[c] // 2
        return [
            pltpu.make_async_copy(
                rbufs.at[c % NBUF, pl.ds(p * h, h)],
                outs[k].at[pl.ds(OFFS[c] + p * h, h)],
                out_sems.at[c % NBUF],
            )
            for p in range(2)
        ]

    for c in range(min(NBUF, NCH)):
        in_copy(c).start()

    s = s_ref[0, 0]
    for c in range(NCH):
        if c >= NBUF:
            for k in range(3):
                for cp in out_copies_for(c - NBUF, k):
                    cp.wait()
        in_copy(c).wait()
        x = ibufs[c % NBUF, : CHUNKS[c]]
        valid = jnp.any(x != 0.0, axis=-1, keepdims=True)
        rbufs[c % NBUF, : CHUNKS[c]] = x + jnp.where(valid, s, 0.0)
        for k in range(3):
            for cp in out_copies_for(c, k):
                cp.start()
        if c + NBUF < NCH:
            in_copy(c + NBUF).start()

    for c in range(NCH - min(NBUF, NCH), NCH):
        for k in range(3):
            for cp in out_copies_for(c, k):
                cp.wait()


@jax.jit
def _run(inputs, shift_s):
    f = pl.pallas_call(
        _tc_body,
        out_shape=[jax.ShapeDtypeStruct((N, D), jnp.float32)] * 3,
        in_specs=[
            pl.BlockSpec(memory_space=pltpu.MemorySpace.HBM),
            pl.BlockSpec(memory_space=pltpu.SMEM),
        ],
        out_specs=[pl.BlockSpec(memory_space=pltpu.MemorySpace.HBM)] * 3,
        scratch_shapes=[
            pltpu.VMEM((NBUF, CMAX, D), jnp.float32),
            pltpu.VMEM((NBUF, CMAX, D), jnp.float32),
            pltpu.SemaphoreType.DMA((NBUF,)),
            pltpu.SemaphoreType.DMA((NBUF,)),
        ],
    )
    return f(inputs, shift_s)


def kernel(inputs, shift):
    o1, o2, o3 = _run(inputs, jnp.reshape(shift, (1, 1)))
    return (o1, o2, o3)
